# Initial kernel scaffold; baseline (speedup 1.0000x reference)
import jax
import jax.numpy as jnp
from jax.experimental import pallas as pl


def kernel(x, edge_index, W1, b1, W2, b2, Wlin, blin):
    raise NotImplementedError("write your pallas kernel here")


# speedup vs baseline: 13.4860x; 13.4860x over previous
import functools

import jax
import jax.numpy as jnp
from jax import lax
from jax.experimental import pallas as pl
from jax.experimental.pallas import tpu as pltpu
from jax.experimental.pallas import tpu_sc as plsc

NC = 2
NS = 16
NW = NC * NS
CH = 128


def _zero_copy_chunks(rows_per_tile):
    chunks = []
    r = 0
    while r < rows_per_tile:
        sz = min(CH, rows_per_tile - r)
        chunks.append((r, sz))
        r += sz
    return chunks


@functools.lru_cache(maxsize=None)
def _sc_propagate(n_pad, k, d, with_gather):
    rows_per_tile = n_pad // NS
    chunks = _zero_copy_chunks(rows_per_tile)
    mesh = plsc.VectorSubcoreMesh(core_axis_name="c", subcore_axis_name="s")

    def body(g_hbm, src_hbm, dst_hbm, const_hbm, out_hbm,
             src_v, dst_v, rows_v, acc_sh, sem):
        cid = lax.axis_index("c")
        sid = lax.axis_index("s")
        wid = cid * NS + sid
        r0 = sid * rows_per_tile
        pltpu.sync_copy(const_hbm.at[pl.ds(0, CH)], rows_v)
        for (off, sz) in chunks:
            pltpu.sync_copy(rows_v.at[pl.ds(0, sz)],
                            acc_sh.at[pl.ds(r0 + off, sz)])
        plsc.subcore_barrier()
        pltpu.sync_copy(src_hbm.at[wid], src_v)
        pltpu.sync_copy(dst_hbm.at[wid], dst_v)

        if with_gather:
            @pl.loop(0, k)
            def _edge_chunk(j):
                pltpu.async_copy(g_hbm.at[src_v.at[j]], rows_v, sem).wait()
                pltpu.sync_copy(rows_v, acc_sh.at[dst_v.at[j]], add=True)
        else:
            pltpu.sync_copy(const_hbm.at[pl.ds(CH, CH)], rows_v)

            @pl.loop(0, k)
            def _edge_chunk(j):
                pltpu.sync_copy(rows_v, acc_sh.at[dst_v.at[j]], add=True)

        plsc.subcore_barrier()
        for (off, sz) in chunks:
            pltpu.sync_copy(acc_sh.at[pl.ds(r0 + off, sz)],
                            rows_v.at[pl.ds(0, sz)])
            pltpu.sync_copy(rows_v.at[pl.ds(0, sz)],
                            out_hbm.at[cid, pl.ds(r0 + off, sz)])

    return pl.kernel(
        body,
        out_type=jax.ShapeDtypeStruct((NC, n_pad, d), jnp.float32),
        mesh=mesh,
        scratch_types=[
            pltpu.VMEM((k, CH), jnp.int32),
            pltpu.VMEM((k, CH), jnp.int32),
            pltpu.VMEM((CH, d), jnp.float32),
            pltpu.VMEM_SHARED((n_pad, d), jnp.float32),
            pltpu.SemaphoreType.DMA,
        ],
        compiler_params=pltpu.CompilerParams(use_tc_tiling_on_sc=False),
        name=f"sc_prop_d{d}_{'gather' if with_gather else 'deg'}",
    )


def _dinv(degp_ref):
    deg = degp_ref[0, :, 0:1] + degp_ref[1, :, 0:1] + 1.0
    return lax.rsqrt(deg)


def _tc_pre_body(degp, x, w1, g0):
    dinv = _dinv(degp)
    g0[...] = jnp.dot(x[...], w1[...],
                      preferred_element_type=jnp.float32) * dinv


def _tc_mid_body(degp, s, g, w2, b1, x1_out, g1_out):
    dinv = _dinv(degp)
    x1 = jnp.maximum(dinv * (s[0] + s[1] + g[...]) + b1[...], 0.0)
    x1_out[...] = x1
    g1_out[...] = jnp.dot(x1, w2[...],
                          preferred_element_type=jnp.float32) * dinv


def _tc_jk_body(degp, s, g1, b2, x1, wl1, wl2, gy_out):
    dinv = _dinv(degp)
    x2 = jnp.maximum(dinv * (s[0] + s[1] + g1[...]) + b2[...], 0.0)
    y = (jnp.dot(x1[...], wl1[...], preferred_element_type=jnp.float32)
         + jnp.dot(x2, wl2[...], preferred_element_type=jnp.float32))
    gy_out[...] = y * dinv


def _tc_out_body(degp, s, gy, blin, out):
    dinv = _dinv(degp)
    out[...] = dinv * (s[0] + s[1] + gy[...]) + blin[...]


def kernel(x, edge_index, W1, b1, W2, b2, Wlin, blin):
    n, f = x.shape
    hid = W1.shape[1]
    ncls = Wlin.shape[1]
    e = edge_index.shape[1]
    n_pad = -(-(n + 1) // (NS * 8)) * (NS * 8)
    k = -(-e // (NW * CH))
    e_pad = NW * k * CH
    rb = 1000
    assert n % rb == 0
    grid = n // rb

    src = edge_index[0]
    dst = edge_index[1]
    pad = e_pad - e
    src3 = jnp.concatenate(
        [src, jnp.zeros((pad,), jnp.int32)]).reshape(NW, k, CH)
    dst3 = jnp.concatenate(
        [dst, jnp.full((pad,), n, jnp.int32)]).reshape(NW, k, CH)

    zeros_h = jnp.zeros((2 * CH, hid), jnp.float32)
    zeros_c = jnp.zeros((2 * CH, ncls), jnp.float32)
    ones16 = jnp.concatenate([jnp.zeros((CH, 16), jnp.float32),
                              jnp.ones((CH, 16), jnp.float32)])
    b1r = b1.reshape(1, hid)
    b2r = b2.reshape(1, hid)
    blinr = blin.reshape(1, ncls)
    wl1 = Wlin[:hid]
    wl2 = Wlin[hid:]

    deg_kernel = _sc_propagate(n_pad, k, 16, False)
    prop_h = _sc_propagate(n_pad, k, hid, True)
    prop_c = _sc_propagate(n_pad, k, ncls, True)

    dummy16 = jnp.zeros((n, 16), jnp.float32)
    degp = deg_kernel(dummy16, src3, dst3, ones16)

    degp_spec = pl.BlockSpec((NC, rb, 16), lambda i: (0, i, 0))
    row_spec_h = pl.BlockSpec((rb, hid), lambda i: (i, 0))
    row_spec_c = pl.BlockSpec((rb, ncls), lambda i: (i, 0))
    s_spec_h = pl.BlockSpec((NC, rb, hid), lambda i: (0, i, 0))
    s_spec_c = pl.BlockSpec((NC, rb, ncls), lambda i: (0, i, 0))
    full = lambda shape: pl.BlockSpec(shape, lambda i: tuple(0 for _ in shape))

    g0 = pl.pallas_call(
        _tc_pre_body,
        grid=(grid,),
        in_specs=[degp_spec, pl.BlockSpec((rb, f), lambda i: (i, 0)),
                  full((f, hid))],
        out_specs=row_spec_h,
        out_shape=jax.ShapeDtypeStruct((n, hid), jnp.float32),
    )(degp, x, W1)

    s0 = prop_h(g0, src3, dst3, zeros_h)

    x1, g1 = pl.pallas_call(
        _tc_mid_body,
        grid=(grid,),
        in_specs=[degp_spec, s_spec_h, row_spec_h, full((hid, hid)),
                  full((1, hid))],
        out_specs=[row_spec_h, row_spec_h],
        out_shape=[jax.ShapeDtypeStruct((n, hid), jnp.float32),
                   jax.ShapeDtypeStruct((n, hid), jnp.float32)],
    )(degp, s0, g0, W2, b1r)

    s1 = prop_h(g1, src3, dst3, zeros_h)

    gy = pl.pallas_call(
        _tc_jk_body,
        grid=(grid,),
        in_specs=[degp_spec, s_spec_h, row_spec_h, full((1, hid)),
                  row_spec_h, full((hid, ncls)), full((hid, ncls))],
        out_specs=row_spec_c,
        out_shape=jax.ShapeDtypeStruct((n, ncls), jnp.float32),
    )(degp, s1, g1, b2r, x1, wl1, wl2)

    s2 = prop_c(gy, src3, dst3, zeros_c)

    out = pl.pallas_call(
        _tc_out_body,
        grid=(grid,),
        in_specs=[degp_spec, s_spec_c, row_spec_c, full((1, ncls))],
        out_specs=row_spec_c,
        out_shape=jax.ShapeDtypeStruct((n, ncls), jnp.float32),
    )(degp, s2, gy, blinr)

    return (out, out)
